# baseline (speedup 1.0000x reference)
import functools

import jax
import jax.numpy as jnp
from jax.experimental import pallas as pl
from jax.experimental.pallas import tpu as pltpu

N_NODES = 10000
N_EDGES = 320000
D_FEAT = 128
N_GRAPHS = 16
D_H = 1024
EPS = 1e-5

ROW_BLK = 1000


def _bdot(a, b):
    return jax.lax.dot_general(a.astype(jnp.bfloat16), b.astype(jnp.bfloat16),
                               (((1,), (0,)), ((), ())),
                               preferred_element_type=jnp.float32)


def _y_body(x_ref, w1_ref, b1_ref, y_ref, cs_ref, cs_acc):
    i = pl.program_id(0)
    nsteps = pl.num_programs(0)
    y = _bdot(x_ref[...], w1_ref[...]) + b1_ref[0:1, :]
    y_ref[...] = y

    @pl.when(i == 0)
    def _():
        cs_acc[...] = jnp.zeros_like(cs_acc)

    cs_acc[...] += jnp.sum(y, axis=0, keepdims=True)

    @pl.when(i == nsteps - 1)
    def _():
        cs_ref[...] = jnp.broadcast_to(cs_acc[...], cs_ref.shape)


def _y_pass(x, W1, b1r):
    return pl.pallas_call(
        _y_body,
        grid=(N_NODES // ROW_BLK,),
        in_specs=[
            pl.BlockSpec((ROW_BLK, D_FEAT), lambda i: (i, 0)),
            pl.BlockSpec((D_FEAT, D_H), lambda i: (0, 0)),
            pl.BlockSpec((1, D_H), lambda i: (0, 0)),
        ],
        out_specs=[
            pl.BlockSpec((ROW_BLK, D_H), lambda i: (i, 0)),
            pl.BlockSpec((8, D_H), lambda i: (0, 0)),
        ],
        out_shape=[
            jax.ShapeDtypeStruct((N_NODES, D_H), jnp.float32),
            jax.ShapeDtypeStruct((8, D_H), jnp.float32),
        ],
        scratch_shapes=[pltpu.VMEM((1, D_H), jnp.float32)],
    )(x, W1, b1r)


def _var_body(y_ref, mean_ref, var_ref, acc):
    i = pl.program_id(0)
    nsteps = pl.num_programs(0)

    @pl.when(i == 0)
    def _():
        acc[...] = jnp.zeros_like(acc)

    d = y_ref[...] - mean_ref[0:1, :]
    acc[...] += jnp.sum(d * d, axis=0, keepdims=True)

    @pl.when(i == nsteps - 1)
    def _():
        var_ref[...] = jnp.broadcast_to(acc[...] / jnp.float32(N_NODES),
                                        var_ref.shape)


def _var_pass(y, mean):
    return pl.pallas_call(
        _var_body,
        grid=(N_NODES // ROW_BLK,),
        in_specs=[
            pl.BlockSpec((ROW_BLK, D_H), lambda i: (i, 0)),
            pl.BlockSpec((1, D_H), lambda i: (0, 0)),
        ],
        out_specs=pl.BlockSpec((8, D_H), lambda i: (0, 0)),
        out_shape=jax.ShapeDtypeStruct((8, D_H), jnp.float32),
        scratch_shapes=[pltpu.VMEM((1, D_H), jnp.float32)],
    )(y, mean)


def _hp_body(y_ref, mean_ref, var_ref, g_ref, be_ref, wp_ref, h_ref, p_ref):
    h = jnp.maximum(
        g_ref[0:1, :] * (y_ref[...] - mean_ref[0:1, :])
        / jnp.sqrt(var_ref[0:1, :] + EPS) + be_ref[0:1, :], 0.0)
    h_ref[...] = h
    p_ref[...] = _bdot(h, wp_ref[...])


def _h_and_p(y, mean, var, g1r, be1r, Wp2):
    return pl.pallas_call(
        _hp_body,
        grid=(N_NODES // ROW_BLK,),
        in_specs=[
            pl.BlockSpec((ROW_BLK, D_H), lambda i: (i, 0)),
            pl.BlockSpec((1, D_H), lambda i: (0, 0)),
            pl.BlockSpec((1, D_H), lambda i: (0, 0)),
            pl.BlockSpec((1, D_H), lambda i: (0, 0)),
            pl.BlockSpec((1, D_H), lambda i: (0, 0)),
            pl.BlockSpec((D_H, 128), lambda i: (0, 0)),
        ],
        out_specs=[
            pl.BlockSpec((ROW_BLK, D_H), lambda i: (i, 0)),
            pl.BlockSpec((ROW_BLK, 128), lambda i: (i, 0)),
        ],
        out_shape=[
            jax.ShapeDtypeStruct((N_NODES, D_H), jnp.float32),
            jax.ShapeDtypeStruct((N_NODES, 128), jnp.float32),
        ],
    )(y, mean, var, g1r, be1r, Wp2)


def _segment_softmax(e, seg, num_segments):
    m = jax.ops.segment_max(e, seg, num_segments=num_segments)
    m = jnp.where(jnp.isfinite(m), m, 0.0)
    ex = jnp.exp(e - m[seg])
    denom = jax.ops.segment_sum(ex, seg, num_segments=num_segments)
    return ex / (denom[seg] + 1e-16)


def _greedy_merge(e, src, dst, num_nodes):
    order = jnp.argsort(-e, stable=True)

    def body(i, state):
        remaining, cluster, scores, c = state
        ei = order[i]
        s = src[ei]
        d = dst[ei]
        take = remaining[s] & remaining[d]
        scores = scores.at[c].set(jnp.where(take, e[ei], scores[c]))
        cluster = cluster.at[s].set(jnp.where(take, c, cluster[s]))
        cluster = cluster.at[d].set(jnp.where(take, c, cluster[d]))
        remaining = remaining.at[s].set(remaining[s] & ~take)
        remaining = remaining.at[d].set(remaining[d] & ~take)
        c = c + take.astype(c.dtype)
        return remaining, cluster, scores, c

    remaining0 = jnp.ones((num_nodes,), dtype=bool)
    cluster0 = jnp.full((num_nodes,), -1, dtype=jnp.int32)
    scores0 = jnp.ones((num_nodes,), dtype=e.dtype)
    c0 = jnp.zeros((), dtype=jnp.int32)
    remaining, cluster, scores, c = jax.lax.fori_loop(
        0, e.shape[0], body, (remaining0, cluster0, scores0, c0))
    rank = jnp.cumsum(remaining.astype(jnp.int32)) - 1
    cluster = jnp.where(remaining, c + rank, cluster)
    num_clusters = c + jnp.sum(remaining.astype(jnp.int32))
    return cluster, scores, num_clusters


def kernel(x, edge_index, batch, W1, b1, g1, be1, Wp, bp, W2, b2, g2, be2,
           Wfc, bfc, Wfc1, bfc1):
    src = edge_index[0]
    dst = edge_index[1]

    b1r = b1.reshape(1, D_H)
    g1r = g1.reshape(1, D_H)
    be1r = be1.reshape(1, D_H)
    y, cs8 = _y_pass(x, W1, b1r)
    mean = (cs8[0:1] / jnp.float32(N_NODES))
    var8 = _var_pass(y, mean)

    Wp2 = jnp.zeros((D_H, 128), jnp.float32)
    Wp2 = Wp2.at[:, 0].set(Wp[:D_H, 0]).at[:, 1].set(Wp[D_H:, 0])
    h, ppad = _h_and_p(y, mean, var8[0:1], g1r, be1r, Wp2)
    p0 = ppad[:, 0]
    p1 = ppad[:, 1]

    e = p0[src] + p1[dst] + bp[0]
    e = _segment_softmax(e, dst, N_NODES) + 0.5
    cluster, scores, C = _greedy_merge(e, src, dst, N_NODES)
    valid = jnp.arange(N_NODES) < C
    new_x = jax.ops.segment_sum(h, cluster, num_segments=N_NODES)
    new_x = new_x * scores[:, None]
    new_batch = jax.ops.segment_max(batch, cluster, num_segments=N_NODES)
    new_batch = jnp.where(valid, new_batch, N_GRAPHS)

    pre = new_x @ W2 + b2
    Cf = C.astype(pre.dtype)
    vmask = valid[:, None]
    mean = jnp.sum(jnp.where(vmask, pre, 0.0), axis=0) / Cf
    var = jnp.sum(jnp.where(vmask, (pre - mean) ** 2, 0.0), axis=0) / Cf
    h2 = jax.nn.relu(g2 * (pre - mean) / jnp.sqrt(var + EPS) + be2)
    sums = jax.ops.segment_sum(h2, new_batch, num_segments=N_GRAPHS)
    counts = jax.ops.segment_sum(jnp.ones((h2.shape[0],), h2.dtype),
                                 new_batch, num_segments=N_GRAPHS)
    pooled = sums / jnp.maximum(counts, 1.0)[:, None]
    z = jax.nn.relu(pooled @ Wfc + bfc)
    return z @ Wfc1 + bfc1
